# baseline (speedup 1.0000x reference)
import functools

import jax
import jax.numpy as jnp
from jax import lax
from jax.experimental import pallas as pl
from jax.experimental.pallas import tpu as pltpu
from jax.experimental.pallas import tpu_sc as plsc

VOCAB = 100000
EMBED_DIM = 128
BATCH = 16384
FIELDS = 26
FPAD = 32

NW = 32
CHUNK = 128
NBUF = 4
NSTAGE = 2
SBATCH = BATCH // NSTAGE
SROWS = SBATCH * FPAD
PER_W = SROWS // NW
NCHUNK = PER_W // CHUNK

BB = 64


def _sc_gather_stage(idx2d, table, stage):
    mesh = plsc.VectorSubcoreMesh(core_axis_name="c", subcore_axis_name="s")

    @functools.partial(
        pl.kernel,
        mesh=mesh,
        out_type=jax.ShapeDtypeStruct((SROWS, EMBED_DIM), jnp.float32),
        scratch_types=[
            pltpu.VMEM((NCHUNK, CHUNK), jnp.int32),
            *[pltpu.VMEM((CHUNK, EMBED_DIM), jnp.float32) for _ in range(NBUF)],
            *[pltpu.SemaphoreType.DMA for _ in range(NBUF)],
            *[pltpu.SemaphoreType.DMA for _ in range(NBUF)],
        ],
    )
    def k(idx_hbm, table_hbm, out_hbm, idx_v,
          buf0, buf1, buf2, buf3, g0, g1, g2, g3, o0, o1, o2, o3):
        bufs = (buf0, buf1, buf2, buf3)
        gsems = (g0, g1, g2, g3)
        osems = (o0, o1, o2, o3)
        wid = lax.axis_index("s") * 2 + lax.axis_index("c")
        row0 = stage * (SROWS // CHUNK) + wid * NCHUNK
        lrow0 = wid * NCHUNK

        pltpu.sync_copy(idx_hbm.at[pl.ds(row0, NCHUNK)], idx_v)

        def gather_start(j, b):
            pltpu.make_async_copy(
                table_hbm.at[idx_v.at[j]], bufs[b], gsems[b]
            ).start()

        def gather_wait(b):
            pltpu.make_async_copy(
                table_hbm.at[idx_v.at[0]], bufs[b], gsems[b]
            ).wait()

        def out_start(j, b):
            pltpu.make_async_copy(
                bufs[b],
                out_hbm.at[pl.ds((lrow0 + j) * CHUNK, CHUNK)],
                osems[b],
            ).start()

        def out_wait(b):
            pltpu.make_async_copy(
                bufs[b],
                out_hbm.at[pl.ds(lrow0 * CHUNK, CHUNK)],
                osems[b],
            ).wait()

        gather_start(0, 0)
        gather_start(1, 1)

        def step(i, _):
            c0 = i * NBUF
            for b in range(NBUF):
                c = c0 + b
                b2 = (b + 2) % NBUF

                @pl.when(c >= 2)
                def _():
                    out_wait(b2)

                @pl.when(c + 2 < NCHUNK)
                def _():
                    gather_start(c + 2, b2)

                gather_wait(b)
                out_start(c, b)
            return 0

        lax.fori_loop(0, NCHUNK // NBUF, step, 0)

        out_wait((NCHUNK - 2) % NBUF)
        out_wait((NCHUNK - 1) % NBUF)

    return k(idx2d, table)


def _tc_reblock_stage(prev, rows3d, stage):
    grid = (SBATCH // BB,)
    base_blk = stage * SBATCH // BB

    def body(*refs):
        x_ref, o_ref = refs[-2], refs[-1]
        o_ref[...] = x_ref[:, :FIELDS, :]

    row_spec = pl.BlockSpec((BB, FPAD, EMBED_DIM), lambda i: (i, 0, 0))
    out_spec = pl.BlockSpec(
        (BB, FIELDS, EMBED_DIM), lambda i: (base_blk + i, 0, 0)
    )
    out_shape = jax.ShapeDtypeStruct((BATCH, FIELDS, EMBED_DIM), jnp.float32)
    if stage == 0:
        return pl.pallas_call(
            body,
            grid=grid,
            in_specs=[row_spec],
            out_specs=out_spec,
            out_shape=out_shape,
        )(rows3d)
    return pl.pallas_call(
        body,
        grid=grid,
        in_specs=[pl.BlockSpec(memory_space=pl.ANY), row_spec],
        out_specs=out_spec,
        out_shape=out_shape,
        input_output_aliases={0: 0},
    )(prev, rows3d)


def kernel(word_idx, embeddings):
    idx_pad = jnp.pad(word_idx.astype(jnp.int32), ((0, 0), (0, FPAD - FIELDS)))
    idx2d = idx_pad.reshape(BATCH * FPAD // CHUNK, CHUNK)
    out = None
    for s in range(NSTAGE):
        rows = _sc_gather_stage(idx2d, embeddings, s)
        out = _tc_reblock_stage(out, rows.reshape(SBATCH, FPAD, EMBED_DIM), s)
    return out


# speedup vs baseline: 8.3232x; 8.3232x over previous
import functools

import jax
import jax.numpy as jnp
from jax import lax
from jax.experimental import pallas as pl
from jax.experimental.pallas import tpu as pltpu
from jax.experimental.pallas import tpu_sc as plsc

VOCAB = 100000
EMBED_DIM = 128
BATCH = 16384
FIELDS = 26

TOTAL = BATCH * FIELDS
NW = 32
BPC = 4
CHUNK = BPC * FIELDS
NBUF = 4
NSTAGE = 2
SBATCH = BATCH // NSTAGE
B_PER_W = SBATCH // NW
NCHUNK = B_PER_W // BPC


def _sc_gather_stage(idx2d, table, stage):
    mesh = plsc.VectorSubcoreMesh(core_axis_name="c", subcore_axis_name="s")

    @functools.partial(
        pl.kernel,
        mesh=mesh,
        out_type=jax.ShapeDtypeStruct((SBATCH, FIELDS, EMBED_DIM), jnp.float32),
        scratch_types=[
            pltpu.VMEM((NCHUNK, CHUNK), jnp.int32),
            *[pltpu.VMEM((CHUNK, EMBED_DIM), jnp.float32) for _ in range(NBUF)],
            *[pltpu.SemaphoreType.DMA for _ in range(NBUF)],
            *[pltpu.SemaphoreType.DMA for _ in range(NBUF)],
        ],
    )
    def k(idx_hbm, table_hbm, out_hbm, idx_v,
          buf0, buf1, buf2, buf3, g0, g1, g2, g3, o0, o1, o2, o3):
        bufs = (buf0, buf1, buf2, buf3)
        gsems = (g0, g1, g2, g3)
        osems = (o0, o1, o2, o3)
        wid = lax.axis_index("s") * 2 + lax.axis_index("c")
        row0 = stage * (SBATCH * FIELDS // CHUNK) + wid * NCHUNK
        b0 = wid * B_PER_W

        pltpu.sync_copy(idx_hbm.at[pl.ds(row0, NCHUNK)], idx_v)

        def gather_start(j, b):
            pltpu.make_async_copy(
                table_hbm.at[idx_v.at[j]], bufs[b], gsems[b]
            ).start()

        def gather_wait(b):
            pltpu.make_async_copy(
                table_hbm.at[idx_v.at[0]], bufs[b], gsems[b]
            ).wait()

        def out_start(j, b):
            for i in range(BPC):
                pltpu.make_async_copy(
                    bufs[b].at[pl.ds(i * FIELDS, FIELDS)],
                    out_hbm.at[b0 + j * BPC + i],
                    osems[b],
                ).start()

        def out_wait(b):
            for _ in range(BPC):
                pltpu.make_async_copy(
                    bufs[b].at[pl.ds(0, FIELDS)],
                    out_hbm.at[b0],
                    osems[b],
                ).wait()

        gather_start(0, 0)
        gather_start(1, 1)

        def step(i, _):
            c0 = i * NBUF
            for b in range(NBUF):
                c = c0 + b
                b2 = (b + 2) % NBUF

                @pl.when(c >= 2)
                def _():
                    out_wait(b2)

                @pl.when(c + 2 < NCHUNK)
                def _():
                    gather_start(c + 2, b2)

                gather_wait(b)
                out_start(c, b)
            return 0

        lax.fori_loop(0, NCHUNK // NBUF, step, 0)

        out_wait((NCHUNK - 2) % NBUF)
        out_wait((NCHUNK - 1) % NBUF)

    return k(idx2d, table)


def kernel(word_idx, embeddings):
    idx2d = word_idx.reshape(TOTAL // CHUNK, CHUNK).astype(jnp.int32)
    pieces = [_sc_gather_stage(idx2d, embeddings, s) for s in range(NSTAGE)]
    out = jnp.zeros((BATCH, FIELDS, EMBED_DIM), jnp.float32)
    for s, piece in enumerate(pieces):
        out = lax.dynamic_update_slice(out, piece, (s * SBATCH, 0, 0))
    return out


# speedup vs baseline: 8.8394x; 1.0620x over previous
import functools

import jax
import jax.numpy as jnp
from jax import lax
from jax.experimental import pallas as pl
from jax.experimental.pallas import tpu as pltpu
from jax.experimental.pallas import tpu_sc as plsc

VOCAB = 100000
EMBED_DIM = 128
BATCH = 16384
FIELDS = 26

TOTAL = BATCH * FIELDS
NW = 32
BPC = 4
CHUNK = BPC * FIELDS
NSTAGE = 2
SBATCH = BATCH // NSTAGE
B_PER_W = SBATCH // NW
NCHUNK = B_PER_W // BPC
NBUF = 4


def _sc_gather(idx2d, table, stage):
    mesh = plsc.VectorSubcoreMesh(core_axis_name="c", subcore_axis_name="s")

    @functools.partial(
        pl.kernel,
        mesh=mesh,
        out_type=jax.ShapeDtypeStruct((SBATCH, FIELDS, EMBED_DIM), jnp.float32),
        scratch_types=[
            pltpu.VMEM((NCHUNK, CHUNK), jnp.int32),
            *[pltpu.VMEM((CHUNK, EMBED_DIM), jnp.float32) for _ in range(NBUF)],
            *[pltpu.SemaphoreType.DMA for _ in range(NBUF)],
            *[pltpu.SemaphoreType.DMA for _ in range(NBUF)],
        ],
    )
    def k(idx_hbm, table_hbm, out_hbm, idx_v,
          buf0, buf1, buf2, buf3, g0, g1, g2, g3, o0, o1, o2, o3):
        bufs = (buf0, buf1, buf2, buf3)
        gsems = (g0, g1, g2, g3)
        osems = (o0, o1, o2, o3)
        wid = lax.axis_index("s") * 2 + lax.axis_index("c")
        row0 = stage * (SBATCH * FIELDS // CHUNK) + wid * NCHUNK
        b0 = wid * B_PER_W

        pltpu.sync_copy(idx_hbm.at[pl.ds(row0, NCHUNK)], idx_v)

        def gather_start(j, b):
            pltpu.make_async_copy(
                table_hbm.at[idx_v.at[j]], bufs[b], gsems[b]
            ).start()

        def gather_wait(b):
            pltpu.make_async_copy(
                table_hbm.at[idx_v.at[0]], bufs[b], gsems[b]
            ).wait()

        def out_start(j, b):
            for i in range(BPC):
                pltpu.make_async_copy(
                    bufs[b].at[pl.ds(i * FIELDS, FIELDS)],
                    out_hbm.at[b0 + j * BPC + i],
                    osems[b],
                ).start()

        def out_wait(b):
            for _ in range(BPC):
                pltpu.make_async_copy(
                    bufs[b].at[pl.ds(0, FIELDS)],
                    out_hbm.at[b0],
                    osems[b],
                ).wait()

        gather_start(0, 0)
        gather_start(1, 1)

        def step(i, _):
            c0 = i * NBUF
            for b in range(NBUF):
                c = c0 + b
                b2 = (b + 2) % NBUF

                @pl.when(c >= 2)
                def _():
                    out_wait(b2)

                @pl.when(c + 2 < NCHUNK)
                def _():
                    gather_start(c + 2, b2)

                gather_wait(b)
                out_start(c, b)
            return 0

        lax.fori_loop(0, NCHUNK // NBUF, step, 0)

        out_wait((NCHUNK - 2) % NBUF)
        out_wait((NCHUNK - 1) % NBUF)

    return k(idx2d, table)


def kernel(word_idx, embeddings):
    idx2d = word_idx.reshape(TOTAL // CHUNK, CHUNK).astype(jnp.int32)
    pieces = [_sc_gather(idx2d, embeddings, s) for s in range(NSTAGE)]
    return jnp.concatenate(pieces, axis=0)


# speedup vs baseline: 13.3894x; 1.5147x over previous
import functools

import jax
import jax.numpy as jnp
from jax import lax
from jax.experimental import pallas as pl
from jax.experimental.pallas import tpu as pltpu
from jax.experimental.pallas import tpu_sc as plsc

VOCAB = 100000
EMBED_DIM = 128
BATCH = 16384
FIELDS = 26

TOTAL = BATCH * FIELDS
NW = 32
BPC = 4
CHUNK = BPC * FIELDS
B_PER_W = BATCH // NW
NCHUNK = B_PER_W // BPC
NBUF = 8
LOOK = NBUF // 2


def _sc_gather(idx2d, table):
    mesh = plsc.VectorSubcoreMesh(core_axis_name="c", subcore_axis_name="s")

    @functools.partial(
        pl.kernel,
        mesh=mesh,
        out_type=jax.ShapeDtypeStruct((BATCH, FIELDS, EMBED_DIM), jnp.float32),
        scratch_types=[
            pltpu.VMEM((NCHUNK, CHUNK), jnp.int32),
            *[pltpu.VMEM((CHUNK, EMBED_DIM), jnp.float32) for _ in range(NBUF)],
            *[pltpu.SemaphoreType.DMA for _ in range(NBUF)],
            *[pltpu.SemaphoreType.DMA for _ in range(NBUF)],
        ],
    )
    def k(idx_hbm, table_hbm, out_hbm, idx_v, *rest):
        bufs = rest[:NBUF]
        gsems = rest[NBUF:2 * NBUF]
        osems = rest[2 * NBUF:3 * NBUF]
        wid = lax.axis_index("s") * 2 + lax.axis_index("c")
        row0 = wid * NCHUNK
        b0 = wid * B_PER_W

        pltpu.sync_copy(idx_hbm.at[pl.ds(row0, NCHUNK)], idx_v)

        def gather_start(j, b):
            pltpu.make_async_copy(
                table_hbm.at[idx_v.at[j]], bufs[b], gsems[b]
            ).start()

        def gather_wait(b):
            pltpu.make_async_copy(
                table_hbm.at[idx_v.at[0]], bufs[b], gsems[b]
            ).wait()

        def out_start(j, b):
            for i in range(BPC):
                pltpu.make_async_copy(
                    bufs[b].at[pl.ds(i * FIELDS, FIELDS)],
                    out_hbm.at[b0 + j * BPC + i],
                    osems[b],
                ).start()

        def out_wait(b):
            for _ in range(BPC):
                pltpu.make_async_copy(
                    bufs[b].at[pl.ds(0, FIELDS)],
                    out_hbm.at[b0],
                    osems[b],
                ).wait()

        for j in range(LOOK):
            gather_start(j, j)

        def step(i, _):
            c0 = i * NBUF
            for b in range(NBUF):
                c = c0 + b
                b2 = (b + LOOK) % NBUF

                @pl.when(c >= LOOK)
                def _():
                    out_wait(b2)

                @pl.when(c + LOOK < NCHUNK)
                def _():
                    gather_start(c + LOOK, b2)

                gather_wait(b)
                out_start(c, b)
            return 0

        lax.fori_loop(0, NCHUNK // NBUF, step, 0)

        for j in range(LOOK):
            out_wait((NCHUNK - LOOK + j) % NBUF)

    return k(idx2d, table)


def kernel(word_idx, embeddings):
    idx2d = word_idx.reshape(TOTAL // CHUNK, CHUNK).astype(jnp.int32)
    return _sc_gather(idx2d, embeddings)
